# baseline (speedup 1.0000x reference)
import functools

import jax
import jax.numpy as jnp
from jax import lax
from jax.experimental import pallas as pl
from jax.experimental.pallas import tpu as pltpu
from jax.experimental.pallas import tpu_sc as plsc

B, L, K, NA = 4, 1024, 30, 22
KP = 32
ROW = K * NA * NA
NC, NS, LANES = 2, 16, 16
NW = NC * NS
RPW = (B * L) // NW
CH = 2
NCH = RPW // CH
EPAD = 1024


def _sc_logits(etab_flat, eidx_flat, seqs_flat):
    mesh = plsc.VectorSubcoreMesh(core_axis_name="c", subcore_axis_name="s",
                                  num_cores=NC, num_subcores=NS)

    @functools.partial(
        pl.kernel,
        out_type=jax.ShapeDtypeStruct((B * L * KP,), jnp.float32),
        mesh=mesh,
        scratch_types=[
            pltpu.VMEM((CH * ROW + EPAD,), jnp.float32),
            pltpu.VMEM((CH * ROW + EPAD,), jnp.float32),
            pltpu.VMEM((RPW * KP,), jnp.int32),
            pltpu.VMEM((RPW * KP,), jnp.int32),
            pltpu.VMEM((L,), jnp.int32),
            pltpu.VMEM((RPW * KP,), jnp.float32),
            pltpu.SemaphoreType.DMA,
            pltpu.SemaphoreType.DMA,
        ],
        compiler_params=pltpu.CompilerParams(needs_layout_passes=False),
    )
    def sc_kernel(etab_h, eidx_h, seqs_h, out_h,
                  ebuf0, ebuf1, eidx_v, c_v, seqs_v, out_v, sem0, sem1):
        cid = lax.axis_index("c")
        sid = lax.axis_index("s")
        wid = sid * NC + cid
        b = wid // (NW // B)
        row0 = wid * RPW

        pltpu.sync_copy(seqs_h.at[pl.ds(b * L, L)], seqs_v)
        pltpu.sync_copy(eidx_h.at[pl.ds(row0 * KP, RPW * KP)], eidx_v)

        def cstage(t, carry):
            ev = eidx_v[pl.ds(t * LANES, LANES)]
            c_v[pl.ds(t * LANES, LANES)] = plsc.load_gather(seqs_v, [ev])
            return carry
        lax.fori_loop(0, (RPW * KP) // LANES, cstage, 0)

        def dma_start(g, buf, sem):
            src = etab_h.at[pl.ds((row0 + g * CH) * ROW, CH * ROW)]
            pltpu.make_async_copy(src, buf.at[pl.ds(0, CH * ROW)], sem).start()

        def dma_wait(buf, sem):
            src = etab_h.at[pl.ds(0, CH * ROW)]
            pltpu.make_async_copy(src, buf.at[pl.ds(0, CH * ROW)], sem).wait()

        iota = lax.iota(jnp.int32, LANES)
        a_lo = iota * NA
        a_hi = (iota + LANES) * NA

        def compute_chunk(g, buf):
            for lr in range(CH):
                r = g * CH + lr
                acc0 = jnp.zeros((LANES,), jnp.float32)
                acc1 = jnp.zeros((LANES,), jnp.float32)
                c_lo = c_v[pl.ds(r * KP, LANES)]
                c_hi = c_v[pl.ds(r * KP + LANES, LANES)]
                for j in range(K):
                    c = c_lo[j] if j < LANES else c_hi[j - LANES]
                    base = lr * ROW + j * (NA * NA) + c
                    acc0 = acc0 + plsc.load_gather(buf, [a_lo + base])
                    acc1 = acc1 + plsc.load_gather(buf, [a_hi + base])
                out_v[pl.ds(r * KP, LANES)] = acc0
                out_v[pl.ds(r * KP + LANES, LANES)] = acc1

        dma_start(0, ebuf0, sem0)
        dma_start(1, ebuf1, sem1)

        def iter_body(i, carry):
            dma_wait(ebuf0, sem0)
            compute_chunk(2 * i, ebuf0)

            @pl.when(i < NCH // 2 - 1)
            def _():
                dma_start(2 * i + 2, ebuf0, sem0)

            dma_wait(ebuf1, sem1)
            compute_chunk(2 * i + 1, ebuf1)

            @pl.when(i < NCH // 2 - 1)
            def _():
                dma_start(2 * i + 3, ebuf1, sem1)

            return carry
        lax.fori_loop(0, NCH // 2, iter_body, 0)

        pltpu.sync_copy(out_v, out_h.at[pl.ds(row0 * KP, RPW * KP)])

    return sc_kernel(etab_flat, eidx_flat, seqs_flat)


def _tc_loss(aa, seqs_col, mask_col):
    def body(aa_ref, seqs_ref, mask_ref, out_ref):
        x = aa_ref[...]
        lane = lax.broadcasted_iota(jnp.int32, (B * L, KP), 1)
        valid = lane < NA
        seqs = seqs_ref[...]
        maskc = mask_ref[...]
        xm = jnp.where(valid, x, -1e30)
        m = jnp.max(xm, axis=1, keepdims=True)
        lse = m + jnp.log(jnp.sum(jnp.exp(xm - m), axis=1, keepdims=True))
        pick = jnp.sum(jnp.where(lane == seqs, x, 0.0), axis=1, keepdims=True)
        logp = (pick - lse) * maskc
        bid = lax.broadcasted_iota(jnp.int32, (B * L, 1), 0) // L
        acc = jnp.float32(0.0)
        for bb in range(B):
            sel = bid == bb
            sb = jnp.sum(jnp.where(sel, logp, 0.0))
            nb = jnp.sum(jnp.where(sel, maskc, 0.0))
            acc = acc + sb / nb
        out_ref[...] = jnp.broadcast_to(-acc / B, (1, 1))

    out = pl.pallas_call(
        body,
        out_shape=jax.ShapeDtypeStruct((1, 1), jnp.float32),
    )(aa, seqs_col, mask_col)
    return out[0, 0]


def kernel(etab, E_idx, seqs, x_mask):
    etab_flat = etab.reshape(-1)
    eidx_flat = jnp.pad(E_idx, ((0, 0), (0, 0), (0, KP - K))).reshape(-1)
    seqs_flat = seqs.reshape(-1)
    aa = _sc_logits(etab_flat, eidx_flat, seqs_flat).reshape(B * L, KP)
    return _tc_loss(aa, seqs_flat.reshape(B * L, 1),
                    x_mask.reshape(B * L, 1).astype(jnp.float32))


# speedup vs baseline: 2.2102x; 2.2102x over previous
import functools

import numpy as np
import jax
import jax.numpy as jnp
from jax import lax
from jax.experimental import pallas as pl
from jax.experimental.pallas import tpu as pltpu
from jax.experimental.pallas import tpu_sc as plsc

B, L, K, NA = 4, 1024, 30, 22
KP = 32
NC, NS, LANES = 2, 16, 16
NW = NC * NS
RPW = (B * L) // NW
BL = 128
GI = L // BL

_SEG = (np.arange(NA * NA)[:, None] // NA == np.arange(NA)[None, :]).astype(
    np.float32)


def _sc_eaa(eidx_flat, seqs_flat):
    mesh = plsc.VectorSubcoreMesh(core_axis_name="c", subcore_axis_name="s",
                                  num_cores=NC, num_subcores=NS)

    @functools.partial(
        pl.kernel,
        out_type=jax.ShapeDtypeStruct((B * L * KP,), jnp.int32),
        mesh=mesh,
        scratch_types=[
            pltpu.VMEM((RPW * KP,), jnp.int32),
            pltpu.VMEM((RPW * KP,), jnp.int32),
            pltpu.VMEM((L,), jnp.int32),
        ],
        compiler_params=pltpu.CompilerParams(needs_layout_passes=False),
    )
    def sck(eidx_h, seqs_h, out_h, eidx_v, c_v, seqs_v):
        cid = lax.axis_index("c")
        sid = lax.axis_index("s")
        wid = sid * NC + cid
        b = wid // (NW // B)
        row0 = wid * RPW

        pltpu.sync_copy(seqs_h.at[pl.ds(b * L, L)], seqs_v)
        pltpu.sync_copy(eidx_h.at[pl.ds(row0 * KP, RPW * KP)], eidx_v)

        def cstage(t, carry):
            ev = eidx_v[pl.ds(t * LANES, LANES)]
            c_v[pl.ds(t * LANES, LANES)] = plsc.load_gather(seqs_v, [ev])
            return carry
        lax.fori_loop(0, (RPW * KP) // LANES, cstage, 0)

        pltpu.sync_copy(c_v, out_h.at[pl.ds(row0 * KP, RPW * KP)])

    return sck(eidx_flat, seqs_flat)


def _tc_main(etab, eaa2d, seqs2d, mask2d, seg):
    def body(etab_ref, eaa_ref, seqs_ref, mask_ref, seg_ref, s_ref, n_ref):
        bb = pl.program_id(0)
        i = pl.program_id(1)
        et3 = etab_ref[0]
        c3 = eaa_ref[:, :K][:, :, None]
        col = lax.broadcasted_iota(jnp.int32, (1, 1, NA * NA), 2) % NA
        masked = jnp.where(col == c3, et3, 0.0)
        s484 = jnp.sum(masked, axis=1)
        aa = jnp.dot(s484, seg_ref[...],
                     preferred_element_type=jnp.float32)
        m = jnp.max(aa, axis=1, keepdims=True)
        lse = m + jnp.log(jnp.sum(jnp.exp(aa - m), axis=1, keepdims=True))
        lane = lax.broadcasted_iota(jnp.int32, (BL, NA), 1)
        pick = jnp.sum(jnp.where(lane == seqs_ref[...], aa, 0.0),
                       axis=1, keepdims=True)
        maskc = mask_ref[...]
        blk_s = jnp.sum((pick - lse) * maskc)
        blk_n = jnp.sum(maskc)

        @pl.when(i == 0)
        def _():
            s_ref[bb, 0] = 0.0
            n_ref[bb, 0] = 0.0

        s_ref[bb, 0] += blk_s
        n_ref[bb, 0] += blk_n

    grid = (B, GI)
    out = pl.pallas_call(
        body,
        grid=grid,
        in_specs=[
            pl.BlockSpec((1, BL, K, NA * NA), lambda b, i: (b, i, 0, 0)),
            pl.BlockSpec((BL, KP), lambda b, i: (b * GI + i, 0)),
            pl.BlockSpec((BL, 1), lambda b, i: (b * GI + i, 0)),
            pl.BlockSpec((BL, 1), lambda b, i: (b * GI + i, 0)),
            pl.BlockSpec((NA * NA, NA), lambda b, i: (0, 0)),
        ],
        out_specs=[
            pl.BlockSpec((B, 1), lambda b, i: (0, 0),
                         memory_space=pltpu.SMEM),
            pl.BlockSpec((B, 1), lambda b, i: (0, 0),
                         memory_space=pltpu.SMEM),
        ],
        out_shape=[
            jax.ShapeDtypeStruct((B, 1), jnp.float32),
            jax.ShapeDtypeStruct((B, 1), jnp.float32),
        ],
    )(etab, eaa2d, seqs2d, mask2d, seg)
    return out


def kernel(etab, E_idx, seqs, x_mask):
    eidx_flat = jnp.pad(E_idx, ((0, 0), (0, 0), (0, KP - K))).reshape(-1)
    seqs_flat = seqs.reshape(-1)
    eaa = _sc_eaa(eidx_flat, seqs_flat)
    s, n = _tc_main(
        etab,
        eaa.reshape(B * L, KP),
        seqs_flat.reshape(B * L, 1),
        x_mask.reshape(B * L, 1).astype(jnp.float32),
        jnp.asarray(_SEG),
    )
    return -jnp.mean(s[:, 0] / n[:, 0])
